# baseline (speedup 1.0000x reference)
import functools

import jax
import jax.numpy as jnp
from jax import lax
from jax.experimental import pallas as pl
from jax.experimental.pallas import tpu as pltpu
from jax.experimental.pallas import tpu_sc as plsc

_N = 100000
_E = 3200000
_D = 128
_NSUB = 16
_NPAD = 100096
_RPT = _NPAD // _NSUB
_G = 1000
_NG = _E // _G
_NGT = _NG // _NSUB
_K = 8
_NB = _NGT // _K


def _proj_body(fu_ref, fi_ref, wr_ref, wv_ref, br_ref, bv_ref, our_ref, ovi_ref):
    our_ref[...] = (
        jnp.dot(fu_ref[...], wr_ref[...], preferred_element_type=jnp.float32)
        + br_ref[...]
    )
    ovi_ref[...] = (
        jnp.dot(fi_ref[...], wv_ref[...], preferred_element_type=jnp.float32)
        + bv_ref[...]
    )


def _project(feat_user, feat_item, W4_rel, W4_rev, b4_rel, b4_rev):
    blk = 1000
    grid = (_N // blk,)
    return pl.pallas_call(
        _proj_body,
        grid=grid,
        in_specs=[
            pl.BlockSpec((blk, _D), lambda i: (i, 0)),
            pl.BlockSpec((blk, _D), lambda i: (i, 0)),
            pl.BlockSpec((_D, 8), lambda i: (0, 0)),
            pl.BlockSpec((_D, 8), lambda i: (0, 0)),
            pl.BlockSpec((1, 8), lambda i: (0, 0)),
            pl.BlockSpec((1, 8), lambda i: (0, 0)),
        ],
        out_specs=[
            pl.BlockSpec((blk, 8), lambda i: (i, 0)),
            pl.BlockSpec((blk, 8), lambda i: (i, 0)),
        ],
        out_shape=[
            jax.ShapeDtypeStruct((_N, 8), jnp.float32),
            jax.ShapeDtypeStruct((_N, 8), jnp.float32),
        ],
    )(feat_user, feat_item, W4_rel, W4_rev, b4_rel, b4_rev)


def _sc_body(whp_rev, whp_rel, ei_rev, ei_rel, zeros_hbm, out_hbm,
             src_v, dst_v, rows_v, acc_sh, gsems):
    c = lax.axis_index("c")
    s = lax.axis_index("s")
    r0 = s * _RPT

    pltpu.sync_copy(zeros_hbm.at[pl.ds(r0, _RPT)], acc_sh.at[pl.ds(r0, _RPT)])
    plsc.subcore_barrier()

    def process(ei, whp):
        base = s * _NGT

        def it(i, carry):
            e0 = (base + i * _K) * _G
            for b in range(_K):
                pltpu.sync_copy(ei.at[0, pl.ds(e0 + b * _G, _G)], src_v.at[b])
                pltpu.sync_copy(ei.at[1, pl.ds(e0 + b * _G, _G)], dst_v.at[b])
            descs = [
                pltpu.async_copy(whp.at[src_v.at[b]], rows_v.at[b], gsems[b])
                for b in range(_K)
            ]
            for b in range(_K):
                descs[b].wait()
                pltpu.sync_copy(rows_v.at[b], acc_sh.at[dst_v.at[b]], add=True)
            return carry
        lax.fori_loop(0, _NB, it, 0)

    @pl.when(c == 0)
    def _():
        process(ei_rev, whp_rev)

    @pl.when(c == 1)
    def _():
        process(ei_rel, whp_rel)

    plsc.subcore_barrier()

    @pl.when(c == 0)
    def _():
        pltpu.sync_copy(acc_sh.at[pl.ds(r0, _RPT)], out_hbm.at[0, pl.ds(r0, _RPT)])

    @pl.when(c == 1)
    def _():
        pltpu.sync_copy(acc_sh.at[pl.ds(r0, _RPT)], out_hbm.at[1, pl.ds(r0, _RPT)])


def _sc_aggregate(whp_rev, whp_rel, ei_rev, ei_rel, zeros):
    mesh = plsc.VectorSubcoreMesh(core_axis_name="c", subcore_axis_name="s")
    f = pl.kernel(
        _sc_body,
        out_type=jax.ShapeDtypeStruct((2, _NPAD, 8), jnp.float32),
        mesh=mesh,
        scratch_types=[
            pltpu.VMEM((_K, _G), jnp.int32),
            pltpu.VMEM((_K, _G), jnp.int32),
            pltpu.VMEM((_K, _G, 8), jnp.float32),
            pltpu.VMEM_SHARED((_NPAD, 8), jnp.float32),
            [pltpu.SemaphoreType.DMA] * _K,
        ],
        compiler_params=pltpu.CompilerParams(use_tc_tiling_on_sc=False),
    )
    return f(whp_rev, whp_rel, ei_rev, ei_rel, zeros)


def _fin_body(acc_ref, out_ref):
    x = acc_ref[...]
    sums = x[:, :, 0:2]
    cnt = x[:, :, 2:3]
    out_ref[...] = jnp.where(cnt > 0.0, sums / jnp.maximum(cnt, 1.0), 0.0)


def _finalize(acc):
    blk = 1000
    grid = (_N // blk,)
    return pl.pallas_call(
        _fin_body,
        grid=grid,
        in_specs=[pl.BlockSpec((2, blk, 8), lambda i: (0, i, 0))],
        out_specs=pl.BlockSpec((2, blk, 2), lambda i: (0, i, 0)),
        out_shape=jax.ShapeDtypeStruct((2, _N, 2), jnp.float32),
    )(acc)


def kernel(feat_user, feat_item, edge_index_rel, edge_index_rev,
           W_rel, b_rel, W_rev, b_rev):
    zpad = jnp.zeros((_D, 6), jnp.float32)
    W4_rel = jnp.concatenate([W_rel, zpad], axis=1)
    W4_rev = jnp.concatenate([W_rev, zpad], axis=1)
    one_zero = jnp.array([1.0, 0.0, 0.0, 0.0, 0.0, 0.0], jnp.float32)
    b4_rel = jnp.concatenate([b_rel, one_zero]).reshape(1, 8)
    b4_rev = jnp.concatenate([b_rev, one_zero]).reshape(1, 8)

    whp_rel, whp_rev = _project(feat_user, feat_item, W4_rel, W4_rev,
                                b4_rel, b4_rev)

    zeros = jnp.zeros((_NPAD, 8), jnp.float32)
    acc = _sc_aggregate(whp_rev, whp_rel, edge_index_rev, edge_index_rel, zeros)
    return _finalize(acc)


# speedup vs baseline: 1.0822x; 1.0822x over previous
import functools

import jax
import jax.numpy as jnp
from jax import lax
from jax.experimental import pallas as pl
from jax.experimental.pallas import tpu as pltpu
from jax.experimental.pallas import tpu_sc as plsc

_N = 100000
_E = 3200000
_D = 128
_NSUB = 16
_NPAD = 100096
_RPT = _NPAD // _NSUB
_G = 2000
_NG = _E // _G
_NGT = _NG // _NSUB
_K = 4
_NB = _NGT // _K


def _proj_body(fu_ref, fi_ref, wr_ref, wv_ref, br_ref, bv_ref, our_ref, ovi_ref):
    our_ref[...] = (
        jnp.dot(fu_ref[...], wr_ref[...], preferred_element_type=jnp.float32)
        + br_ref[...]
    )
    ovi_ref[...] = (
        jnp.dot(fi_ref[...], wv_ref[...], preferred_element_type=jnp.float32)
        + bv_ref[...]
    )


def _project(feat_user, feat_item, W4_rel, W4_rev, b4_rel, b4_rev):
    blk = 1000
    grid = (_N // blk,)
    return pl.pallas_call(
        _proj_body,
        grid=grid,
        in_specs=[
            pl.BlockSpec((blk, _D), lambda i: (i, 0)),
            pl.BlockSpec((blk, _D), lambda i: (i, 0)),
            pl.BlockSpec((_D, 8), lambda i: (0, 0)),
            pl.BlockSpec((_D, 8), lambda i: (0, 0)),
            pl.BlockSpec((1, 8), lambda i: (0, 0)),
            pl.BlockSpec((1, 8), lambda i: (0, 0)),
        ],
        out_specs=[
            pl.BlockSpec((blk, 8), lambda i: (i, 0)),
            pl.BlockSpec((blk, 8), lambda i: (i, 0)),
        ],
        out_shape=[
            jax.ShapeDtypeStruct((_N, 8), jnp.float32),
            jax.ShapeDtypeStruct((_N, 8), jnp.float32),
        ],
    )(feat_user, feat_item, W4_rel, W4_rev, b4_rel, b4_rev)


def _sc_body(whp_rev, whp_rel, ei_rev, ei_rel, zeros_hbm, out_hbm,
             src_v, dst_v, rows_v, acc_sh, gsems):
    c = lax.axis_index("c")
    s = lax.axis_index("s")
    r0 = s * _RPT

    pltpu.sync_copy(zeros_hbm.at[pl.ds(r0, _RPT)], acc_sh.at[pl.ds(r0, _RPT)])
    plsc.subcore_barrier()

    def process(ei, whp):
        base = s * _NGT

        def it(i, carry):
            e0 = (base + i * _K) * _G
            for b in range(_K):
                pltpu.sync_copy(ei.at[0, pl.ds(e0 + b * _G, _G)], src_v.at[b])
                pltpu.sync_copy(ei.at[1, pl.ds(e0 + b * _G, _G)], dst_v.at[b])
            descs = [
                pltpu.async_copy(whp.at[src_v.at[b]], rows_v.at[b], gsems[b])
                for b in range(_K)
            ]
            for b in range(_K):
                descs[b].wait()
                pltpu.sync_copy(rows_v.at[b], acc_sh.at[dst_v.at[b]], add=True)
            return carry
        lax.fori_loop(0, _NB, it, 0)

    @pl.when(c == 0)
    def _():
        process(ei_rev, whp_rev)

    @pl.when(c == 1)
    def _():
        process(ei_rel, whp_rel)

    plsc.subcore_barrier()

    @pl.when(c == 0)
    def _():
        pltpu.sync_copy(acc_sh.at[pl.ds(r0, _RPT)], out_hbm.at[0, pl.ds(r0, _RPT)])

    @pl.when(c == 1)
    def _():
        pltpu.sync_copy(acc_sh.at[pl.ds(r0, _RPT)], out_hbm.at[1, pl.ds(r0, _RPT)])


def _sc_aggregate(whp_rev, whp_rel, ei_rev, ei_rel, zeros):
    mesh = plsc.VectorSubcoreMesh(core_axis_name="c", subcore_axis_name="s")
    f = pl.kernel(
        _sc_body,
        out_type=jax.ShapeDtypeStruct((2, _NPAD, 8), jnp.float32),
        mesh=mesh,
        scratch_types=[
            pltpu.VMEM((_K, _G), jnp.int32),
            pltpu.VMEM((_K, _G), jnp.int32),
            pltpu.VMEM((_K, _G, 8), jnp.float32),
            pltpu.VMEM_SHARED((_NPAD, 8), jnp.float32),
            [pltpu.SemaphoreType.DMA] * _K,
        ],
        compiler_params=pltpu.CompilerParams(use_tc_tiling_on_sc=False),
    )
    return f(whp_rev, whp_rel, ei_rev, ei_rel, zeros)


def _fin_body(acc_ref, out_ref):
    x = acc_ref[...]
    sums = x[:, :, 0:2]
    cnt = x[:, :, 2:3]
    out_ref[...] = jnp.where(cnt > 0.0, sums / jnp.maximum(cnt, 1.0), 0.0)


def _finalize(acc):
    blk = 1000
    grid = (_N // blk,)
    return pl.pallas_call(
        _fin_body,
        grid=grid,
        in_specs=[pl.BlockSpec((2, blk, 8), lambda i: (0, i, 0))],
        out_specs=pl.BlockSpec((2, blk, 2), lambda i: (0, i, 0)),
        out_shape=jax.ShapeDtypeStruct((2, _N, 2), jnp.float32),
    )(acc)


def kernel(feat_user, feat_item, edge_index_rel, edge_index_rev,
           W_rel, b_rel, W_rev, b_rev):
    zpad = jnp.zeros((_D, 6), jnp.float32)
    W4_rel = jnp.concatenate([W_rel, zpad], axis=1)
    W4_rev = jnp.concatenate([W_rev, zpad], axis=1)
    one_zero = jnp.array([1.0, 0.0, 0.0, 0.0, 0.0, 0.0], jnp.float32)
    b4_rel = jnp.concatenate([b_rel, one_zero]).reshape(1, 8)
    b4_rev = jnp.concatenate([b_rev, one_zero]).reshape(1, 8)

    whp_rel, whp_rev = _project(feat_user, feat_item, W4_rel, W4_rev,
                                b4_rel, b4_rev)

    zeros = jnp.zeros((_NPAD, 8), jnp.float32)
    acc = _sc_aggregate(whp_rev, whp_rel, edge_index_rev, edge_index_rel, zeros)
    return _finalize(acc)
